# Initial kernel scaffold; baseline (speedup 1.0000x reference)
import jax
import jax.numpy as jnp
from jax.experimental import pallas as pl


def kernel(user_emb, item_emb, W_map, b_map, ui_edge_index, uu_edge_index, batch_user_pos_neg):
    raise NotImplementedError("write your pallas kernel here")


# speedup vs baseline: 3.7815x; 3.7815x over previous
import functools

import jax
import jax.numpy as jnp
from jax import lax
from jax.experimental import pallas as pl
from jax.experimental.pallas import tpu as pltpu
from jax.experimental.pallas import tpu_sc as plsc

NU = 10000
D = 128
DW = 136
B = 4096

NC = 2
NS = 16
L = 16

NCH_UI = 157
EPAD_UI = NCH_UI * 128 * NS
NCH_UU = 79
EPAD_UU = NCH_UU * 128 * NS * NC

RSZ = 10016

OFF_UU = 4128
OFF_UU2 = 8224
ACC_ROWS = 12416
TRASH_U = 4096
TRASH_I = 8192


def _sc_mesh():
    return plsc.VectorSubcoreMesh(
        core_axis_name="c", subcore_axis_name="s", num_cores=NC, num_subcores=NS
    )


def _sc_body(ua, ia, uiud, uius, uiid, uiis, uud, uus, bu, bp, bn, pos,
             initu, initi, zrows,
             o_uego, o_uui, o_uuu1, o_uuu2, o_ip, o_in, o_isp, o_isn,
             ACC, RSTGU, RSTGI, rows, dbuf, srcb, sbuf, posb, sem):
    c = lax.axis_index("c")
    s = lax.axis_index("s")

    def add_off(off):
        for j in range(128 // L):
            sbuf[pl.ds(j * L, L)] = sbuf[pl.ds(j * L, L)] + off

    pltpu.sync_copy(zrows, ACC.at[pl.ds(s * 776, 776)])

    @pl.when(s == 0)
    def _init_user_remap():
        pltpu.sync_copy(initu, RSTGU)

    @pl.when(jnp.logical_and(c == 1, s == 1))
    def _init_item_remap():
        pltpu.sync_copy(initi, RSTGI)

    plsc.subcore_barrier()

    @pl.when(s == 0)
    def _build_user_remap():
        def body(g, _):
            pltpu.sync_copy(bu.at[pl.ds(g * 128, 128)], dbuf)
            pltpu.sync_copy(pos.at[pl.ds(g * 128, 128)], posb)
            pltpu.sync_copy(posb, RSTGU.at[dbuf])
            return 0
        lax.fori_loop(0, B // 128, body, 0)

    @pl.when(jnp.logical_and(c == 1, s == 1))
    def _build_item_remap():
        def body_p(g, _):
            pltpu.sync_copy(bp.at[pl.ds(g * 128, 128)], dbuf)
            pltpu.sync_copy(pos.at[pl.ds(g * 128, 128)], posb)
            pltpu.sync_copy(posb, RSTGI.at[dbuf])
            return 0
        lax.fori_loop(0, B // 128, body_p, 0)

        def body_n(g, _):
            pltpu.sync_copy(bn.at[pl.ds(g * 128, 128)], dbuf)
            pltpu.sync_copy(pos.at[pl.ds(B + g * 128, 128)], posb)
            pltpu.sync_copy(posb, RSTGI.at[dbuf])
            return 0
        lax.fori_loop(0, B // 128, body_n, 0)

    plsc.subcore_barrier()

    def edge_pass(dst_hbm, src_hbm, table_hbm, rstg, base, nch, slot_off):
        def body(g, _):
            off = base + g * 128
            pltpu.sync_copy(dst_hbm.at[pl.ds(off, 128)], dbuf)
            pltpu.sync_copy(src_hbm.at[pl.ds(off, 128)], srcb)
            pltpu.sync_copy(rstg.at[dbuf], sbuf)
            if slot_off is not None:
                add_off(slot_off)
            pltpu.async_copy(table_hbm.at[srcb], rows, sem).wait()
            pltpu.sync_copy(rows, ACC.at[sbuf], add=True)
            return 0
        lax.fori_loop(0, nch, body, 0)

    @pl.when(c == 0)
    def _ui_user_pass():
        edge_pass(uiud, uius, ia, RSTGU, s * (NCH_UI * 128), NCH_UI, None)

    @pl.when(c == 1)
    def _ui_item_pass():
        edge_pass(uiid, uiis, ua, RSTGI, s * (NCH_UI * 128), NCH_UI, None)

    uu_base = c * (EPAD_UU // NC) + s * (NCH_UU * 128)
    uu_off = jnp.where(c == 0, OFF_UU, OFF_UU2).astype(jnp.int32)
    edge_pass(uud, uus, ua, RSTGU, uu_base, NCH_UU, uu_off)

    plsc.subcore_barrier()

    @pl.when(c == 0)
    def _final_user_side():
        def fin(k, _):
            o = s * 256 + k * 128
            pltpu.sync_copy(bu.at[pl.ds(o, 128)], dbuf)
            pltpu.async_copy(ua.at[dbuf], rows, sem).wait()
            pltpu.sync_copy(rows, o_uego.at[pl.ds(o, 128)])
            pltpu.sync_copy(RSTGU.at[dbuf], sbuf)
            pltpu.sync_copy(ACC.at[sbuf], rows)
            pltpu.sync_copy(rows, o_uui.at[pl.ds(o, 128)])
            add_off(OFF_UU)
            pltpu.sync_copy(ACC.at[sbuf], rows)
            pltpu.sync_copy(rows, o_uuu1.at[pl.ds(o, 128)])
            return 0
        lax.fori_loop(0, 2, fin, 0)

    @pl.when(c == 1)
    def _final_item_side():
        def fin(k, _):
            o = s * 256 + k * 128
            pltpu.sync_copy(bu.at[pl.ds(o, 128)], dbuf)
            pltpu.sync_copy(RSTGU.at[dbuf], sbuf)
            add_off(OFF_UU2)
            pltpu.sync_copy(ACC.at[sbuf], rows)
            pltpu.sync_copy(rows, o_uuu2.at[pl.ds(o, 128)])
            pltpu.sync_copy(bp.at[pl.ds(o, 128)], dbuf)
            pltpu.async_copy(ia.at[dbuf], rows, sem).wait()
            pltpu.sync_copy(rows, o_ip.at[pl.ds(o, 128)])
            pltpu.sync_copy(RSTGI.at[dbuf], sbuf)
            pltpu.sync_copy(ACC.at[sbuf], rows)
            pltpu.sync_copy(rows, o_isp.at[pl.ds(o, 128)])
            pltpu.sync_copy(bn.at[pl.ds(o, 128)], dbuf)
            pltpu.async_copy(ia.at[dbuf], rows, sem).wait()
            pltpu.sync_copy(rows, o_in.at[pl.ds(o, 128)])
            pltpu.sync_copy(RSTGI.at[dbuf], sbuf)
            pltpu.sync_copy(ACC.at[sbuf], rows)
            pltpu.sync_copy(rows, o_isn.at[pl.ds(o, 128)])
            return 0
        lax.fori_loop(0, 2, fin, 0)


_sc_call = functools.partial(
    pl.kernel,
    out_type=[jax.ShapeDtypeStruct((B, DW), jnp.float32)] * 8,
    mesh=_sc_mesh(),
    compiler_params=pltpu.CompilerParams(
        needs_layout_passes=False, use_tc_tiling_on_sc=False),
    scratch_types=[
        pltpu.VMEM_SHARED((ACC_ROWS, DW), jnp.float32),
        pltpu.VMEM_SHARED((RSZ,), jnp.int32),
        pltpu.VMEM_SHARED((RSZ,), jnp.int32),
        pltpu.VMEM((128, DW), jnp.float32),
        pltpu.VMEM((128,), jnp.int32),
        pltpu.VMEM((128,), jnp.int32),
        pltpu.VMEM((128,), jnp.int32),
        pltpu.VMEM((128,), jnp.int32),
        pltpu.SemaphoreType.DMA,
    ],
)(_sc_body)


def _tc_body(ue, uui, du, uu1, uu2, duu1, duu2, ipr, inr, isp, dp, isn, dn,
             W0, W1, W2, bm, out):
    f32 = jnp.float32
    ue_ = ue[...]
    un = uui[...] / jnp.maximum(du[...], 1.0)
    uu = (uu1[...] + uu2[...]) / jnp.maximum(duu1[...] + duu2[...], 1.0)
    h = (
        jnp.dot(ue_, W0[...], preferred_element_type=f32)
        + jnp.dot(un, W1[...], preferred_element_type=f32)
        + jnp.dot(uu, W2[...], preferred_element_type=f32)
        + bm[...]
    )
    u = jnp.tanh(h)
    ipr_ = ipr[...]
    inr_ = inr[...]
    p = 0.5 * (isp[...] / jnp.maximum(dp[...], 1.0)) + 0.5 * ipr_
    n = 0.5 * (isn[...] / jnp.maximum(dn[...], 1.0)) + 0.5 * inr_
    diff = jnp.sum(u * n, axis=-1) - jnp.sum(u * p, axis=-1)
    cf = jnp.mean(jnp.maximum(diff, 0.0) + jnp.log1p(jnp.exp(-jnp.abs(diff))))
    reg = 0.5 * jnp.mean(
        jnp.sum(ue_ * ue_, axis=-1)
        + jnp.sum(ipr_ * ipr_, axis=-1)
        + jnp.sum(inr_ * inr_, axis=-1)
    )
    out[...] = jnp.reshape(1.0 * cf + 1e-4 * reg, (1, 1))


def kernel(user_emb, item_emb, W_map, b_map, ui_edge_index, uu_edge_index,
           batch_user_pos_neg):
    f32 = jnp.float32
    i32 = jnp.int32

    ones_col = jnp.ones((NU, 1), f32)
    pad_cols = jnp.zeros((NU, DW - D - 1), f32)
    ua = jnp.concatenate([user_emb, ones_col, pad_cols], axis=1)
    ia = jnp.concatenate([item_emb, ones_col, pad_cols], axis=1)

    ui_u = ui_edge_index[0].astype(i32)
    ui_i = ui_edge_index[1].astype(i32)
    uu_s = uu_edge_index[0].astype(i32)
    uu_d = uu_edge_index[1].astype(i32)

    def pad_to(x, n, fill):
        return jnp.concatenate(
            [x, jnp.full((n - x.shape[0],), fill, i32)])

    uiud = pad_to(ui_u, EPAD_UI, NU)
    uius = pad_to(ui_i, EPAD_UI, 0)
    uiid = pad_to(ui_i, EPAD_UI, NU)
    uiis = pad_to(ui_u, EPAD_UI, 0)
    uud = pad_to(uu_s, EPAD_UU, NU)
    uus = pad_to(uu_d, EPAD_UU, 0)

    bu = batch_user_pos_neg[:, 0].astype(i32)
    bp = batch_user_pos_neg[:, 1].astype(i32)
    bn = batch_user_pos_neg[:, 2].astype(i32)

    pos = jnp.arange(2 * B, dtype=i32)
    ar = jnp.arange(RSZ, dtype=i32)
    initu = TRASH_U + (ar & 31)
    initi = TRASH_I + (ar & 31)
    zrows = jnp.zeros((776, DW), f32)

    (uego_a, uui_a, uuu1_a, uuu2_a, ip_a, in_a, isp_a, isn_a) = _sc_call(
        ua, ia, uiud, uius, uiid, uiis, uud, uus, bu, bp, bn, pos,
        initu, initi, zrows)

    def split(a):
        return a[:, :D], a[:, D:D + 1]

    ue_, _ = split(uego_a)
    uui_, du = split(uui_a)
    uu1_, duu1 = split(uuu1_a)
    uu2_, duu2 = split(uuu2_a)
    ipr_, _ = split(ip_a)
    inr_, _ = split(in_a)
    isp_, dp = split(isp_a)
    isn_, dn = split(isn_a)

    W0 = W_map[:D]
    W1 = W_map[D:2 * D]
    W2 = W_map[2 * D:]
    bm = b_map.reshape(1, D)

    out = pl.pallas_call(
        _tc_body,
        out_shape=jax.ShapeDtypeStruct((1, 1), f32),
    )(ue_, uui_, du, uu1_, uu2_, duu1, duu2, ipr_, inr_, isp_, dp, isn_, dn,
      W0, W1, W2, bm)
    return out[0, 0]


# speedup vs baseline: 4.5912x; 1.2141x over previous
import functools

import jax
import jax.numpy as jnp
from jax import lax
from jax.experimental import pallas as pl
from jax.experimental.pallas import tpu as pltpu
from jax.experimental.pallas import tpu_sc as plsc

NU = 10000
D = 128
DW = 136
B = 4096

NC = 2
NS = 16
L = 16

E = 320000
NCH_UI = 156
NCH_UU = 78

RSZ = 10016

OFF_UU = 4160
OFF_UU2 = 8256
ACC_ROWS = 12416
TRASH_U = 4096
TRASH_I = 8192


def _sc_mesh():
    return plsc.VectorSubcoreMesh(
        core_axis_name="c", subcore_axis_name="s", num_cores=NC, num_subcores=NS
    )


def _sc_body(ua, ia, ui_ei, uu_ei, bu, bp, bn, pos,
             initu, initi, zrows,
             o_uego, o_uui, o_uuu1, o_uuu2, o_ip, o_in, o_isp, o_isn,
             ACC, RSTGU, RSTGI, rows, dbuf, srcb, sbuf, posb, sem):
    c = lax.axis_index("c")
    s = lax.axis_index("s")

    def add_off(off):
        for j in range(128 // L):
            sbuf[pl.ds(j * L, L)] = sbuf[pl.ds(j * L, L)] + off

    pltpu.sync_copy(zrows, ACC.at[pl.ds(s * 776, 776)])

    @pl.when(s == 0)
    def _init_user_remap():
        pltpu.sync_copy(initu, RSTGU)

    @pl.when(jnp.logical_and(c == 1, s == 1))
    def _init_item_remap():
        pltpu.sync_copy(initi, RSTGI)

    plsc.subcore_barrier()

    @pl.when(s == 0)
    def _build_user_remap():
        def body(g, _):
            pltpu.sync_copy(bu.at[pl.ds(g * 128, 128)], dbuf)
            pltpu.sync_copy(pos.at[pl.ds(g * 128, 128)], posb)
            pltpu.sync_copy(posb, RSTGU.at[dbuf])
            return 0
        lax.fori_loop(0, B // 128, body, 0)

    @pl.when(jnp.logical_and(c == 1, s == 1))
    def _build_item_remap():
        def body_p(g, _):
            pltpu.sync_copy(bp.at[pl.ds(g * 128, 128)], dbuf)
            pltpu.sync_copy(pos.at[pl.ds(g * 128, 128)], posb)
            pltpu.sync_copy(posb, RSTGI.at[dbuf])
            return 0
        lax.fori_loop(0, B // 128, body_p, 0)

        def body_n(g, _):
            pltpu.sync_copy(bn.at[pl.ds(g * 128, 128)], dbuf)
            pltpu.sync_copy(pos.at[pl.ds(B + g * 128, 128)], posb)
            pltpu.sync_copy(posb, RSTGI.at[dbuf])
            return 0
        lax.fori_loop(0, B // 128, body_n, 0)

    plsc.subcore_barrier()

    def chunk(edges, drow, srow, table_hbm, rstg, off, slot_off):
        pltpu.sync_copy(edges.at[drow, pl.ds(off, 128)], dbuf)
        pltpu.sync_copy(edges.at[srow, pl.ds(off, 128)], srcb)
        pltpu.sync_copy(rstg.at[dbuf], sbuf)
        if slot_off is not None:
            add_off(slot_off)
        pltpu.async_copy(table_hbm.at[srcb], rows, sem).wait()
        pltpu.sync_copy(rows, ACC.at[sbuf], add=True)

    def edge_pass(edges, drow, srow, table_hbm, rstg, base, nch, extra,
                  nextra, slot_off):
        def body(g, _):
            chunk(edges, drow, srow, table_hbm, rstg, base + g * 128,
                  slot_off)
            return 0
        lax.fori_loop(0, nch, body, 0)

        @pl.when(s < nextra)
        def _tail():
            chunk(edges, drow, srow, table_hbm, rstg, extra + s * 128,
                  slot_off)

    @pl.when(c == 0)
    def _ui_user_pass():
        edge_pass(ui_ei, 0, 1, ia, RSTGU, s * (NCH_UI * 128), NCH_UI,
                  NS * NCH_UI * 128, 4, None)

    @pl.when(c == 1)
    def _ui_item_pass():
        edge_pass(ui_ei, 1, 0, ua, RSTGI, s * (NCH_UI * 128), NCH_UI,
                  NS * NCH_UI * 128, 4, None)

    uu_half = c * (E // NC)
    uu_off = jnp.where(c == 0, OFF_UU, OFF_UU2).astype(jnp.int32)
    edge_pass(uu_ei, 0, 1, ua, RSTGU, uu_half + s * (NCH_UU * 128), NCH_UU,
              uu_half + NS * NCH_UU * 128, 2, uu_off)

    plsc.subcore_barrier()

    @pl.when(c == 0)
    def _final_user_side():
        def fin(k, _):
            o = s * 256 + k * 128
            pltpu.sync_copy(bu.at[pl.ds(o, 128)], dbuf)
            pltpu.async_copy(ua.at[dbuf], rows, sem).wait()
            pltpu.sync_copy(rows, o_uego.at[pl.ds(o, 128)])
            pltpu.sync_copy(RSTGU.at[dbuf], sbuf)
            pltpu.sync_copy(ACC.at[sbuf], rows)
            pltpu.sync_copy(rows, o_uui.at[pl.ds(o, 128)])
            add_off(OFF_UU)
            pltpu.sync_copy(ACC.at[sbuf], rows)
            pltpu.sync_copy(rows, o_uuu1.at[pl.ds(o, 128)])
            return 0
        lax.fori_loop(0, 2, fin, 0)

    @pl.when(c == 1)
    def _final_item_side():
        def fin(k, _):
            o = s * 256 + k * 128
            pltpu.sync_copy(bu.at[pl.ds(o, 128)], dbuf)
            pltpu.sync_copy(RSTGU.at[dbuf], sbuf)
            add_off(OFF_UU2)
            pltpu.sync_copy(ACC.at[sbuf], rows)
            pltpu.sync_copy(rows, o_uuu2.at[pl.ds(o, 128)])
            pltpu.sync_copy(bp.at[pl.ds(o, 128)], dbuf)
            pltpu.async_copy(ia.at[dbuf], rows, sem).wait()
            pltpu.sync_copy(rows, o_ip.at[pl.ds(o, 128)])
            pltpu.sync_copy(RSTGI.at[dbuf], sbuf)
            pltpu.sync_copy(ACC.at[sbuf], rows)
            pltpu.sync_copy(rows, o_isp.at[pl.ds(o, 128)])
            pltpu.sync_copy(bn.at[pl.ds(o, 128)], dbuf)
            pltpu.async_copy(ia.at[dbuf], rows, sem).wait()
            pltpu.sync_copy(rows, o_in.at[pl.ds(o, 128)])
            pltpu.sync_copy(RSTGI.at[dbuf], sbuf)
            pltpu.sync_copy(ACC.at[sbuf], rows)
            pltpu.sync_copy(rows, o_isn.at[pl.ds(o, 128)])
            return 0
        lax.fori_loop(0, 2, fin, 0)


_sc_call = functools.partial(
    pl.kernel,
    out_type=[jax.ShapeDtypeStruct((B, DW), jnp.float32)] * 8,
    mesh=_sc_mesh(),
    compiler_params=pltpu.CompilerParams(
        needs_layout_passes=False, use_tc_tiling_on_sc=False),
    scratch_types=[
        pltpu.VMEM_SHARED((ACC_ROWS, DW), jnp.float32),
        pltpu.VMEM_SHARED((RSZ,), jnp.int32),
        pltpu.VMEM_SHARED((RSZ,), jnp.int32),
        pltpu.VMEM((128, DW), jnp.float32),
        pltpu.VMEM((128,), jnp.int32),
        pltpu.VMEM((128,), jnp.int32),
        pltpu.VMEM((128,), jnp.int32),
        pltpu.VMEM((128,), jnp.int32),
        pltpu.SemaphoreType.DMA,
    ],
)(_sc_body)


def _tc_body(ue, uui, du, uu1, uu2, duu1, duu2, ipr, inr, isp, dp, isn, dn,
             W0, W1, W2, bm, out):
    f32 = jnp.float32
    ue_ = ue[...]
    un = uui[...] / jnp.maximum(du[...], 1.0)
    uu = (uu1[...] + uu2[...]) / jnp.maximum(duu1[...] + duu2[...], 1.0)
    h = (
        jnp.dot(ue_, W0[...], preferred_element_type=f32)
        + jnp.dot(un, W1[...], preferred_element_type=f32)
        + jnp.dot(uu, W2[...], preferred_element_type=f32)
        + bm[...]
    )
    u = jnp.tanh(h)
    ipr_ = ipr[...]
    inr_ = inr[...]
    p = 0.5 * (isp[...] / jnp.maximum(dp[...], 1.0)) + 0.5 * ipr_
    n = 0.5 * (isn[...] / jnp.maximum(dn[...], 1.0)) + 0.5 * inr_
    diff = jnp.sum(u * n, axis=-1) - jnp.sum(u * p, axis=-1)
    cf = jnp.mean(jnp.maximum(diff, 0.0) + jnp.log1p(jnp.exp(-jnp.abs(diff))))
    reg = 0.5 * jnp.mean(
        jnp.sum(ue_ * ue_, axis=-1)
        + jnp.sum(ipr_ * ipr_, axis=-1)
        + jnp.sum(inr_ * inr_, axis=-1)
    )
    out[...] = jnp.reshape(1.0 * cf + 1e-4 * reg, (1, 1))


def kernel(user_emb, item_emb, W_map, b_map, ui_edge_index, uu_edge_index,
           batch_user_pos_neg):
    f32 = jnp.float32
    i32 = jnp.int32

    ones_col = jnp.ones((NU, 1), f32)
    pad_cols = jnp.zeros((NU, DW - D - 1), f32)
    ua = jnp.concatenate([user_emb, ones_col, pad_cols], axis=1)
    ia = jnp.concatenate([item_emb, ones_col, pad_cols], axis=1)

    ui_ei = ui_edge_index.astype(i32)
    uu_ei = uu_edge_index.astype(i32)

    bu = batch_user_pos_neg[:, 0].astype(i32)
    bp = batch_user_pos_neg[:, 1].astype(i32)
    bn = batch_user_pos_neg[:, 2].astype(i32)

    pos = jnp.arange(2 * B, dtype=i32)
    ar = jnp.arange(RSZ, dtype=i32)
    initu = TRASH_U + (ar & 63)
    initi = TRASH_I + (ar & 63)
    zrows = jnp.zeros((776, DW), f32)

    (uego_a, uui_a, uuu1_a, uuu2_a, ip_a, in_a, isp_a, isn_a) = _sc_call(
        ua, ia, ui_ei, uu_ei, bu, bp, bn, pos, initu, initi, zrows)

    def split(a):
        return a[:, :D], a[:, D:D + 1]

    ue_, _ = split(uego_a)
    uui_, du = split(uui_a)
    uu1_, duu1 = split(uuu1_a)
    uu2_, duu2 = split(uuu2_a)
    ipr_, _ = split(ip_a)
    inr_, _ = split(in_a)
    isp_, dp = split(isp_a)
    isn_, dn = split(isn_a)

    W0 = W_map[:D]
    W1 = W_map[D:2 * D]
    W2 = W_map[2 * D:]
    bm = b_map.reshape(1, D)

    out = pl.pallas_call(
        _tc_body,
        out_shape=jax.ShapeDtypeStruct((1, 1), f32),
    )(ue_, uui_, du, uu1_, uu2_, duu1, duu2, ipr_, inr_, isp_, dp, isn_, dn,
      W0, W1, W2, bm)
    return out[0, 0]


# speedup vs baseline: 6.2184x; 1.3544x over previous
import functools

import jax
import jax.numpy as jnp
from jax import lax
from jax.experimental import pallas as pl
from jax.experimental.pallas import tpu as pltpu
from jax.experimental.pallas import tpu_sc as plsc

NU = 10000
D = 128
DW = 136
B = 4096

NC = 2
NS = 16
L = 16

E = 320000
NCH_UI = 250
NCH_UU = 125

RSZ = 10016

OFF_UU = 4160
OFF_UU2 = 8256
ACC_ROWS = 12416
TRASH_U = 4096
TRASH_I = 8192


def _sc_mesh():
    return plsc.VectorSubcoreMesh(
        core_axis_name="c", subcore_axis_name="s", num_cores=NC, num_subcores=NS
    )


def _sc_body(ua, ia, ui_ei, uu_ei, bu, bp, bn, pos,
             initu, initi, zrows,
             o_uego, o_uui, o_uuu1, o_uuu2, o_ip, o_in, o_isp, o_isn,
             ACC, RSTGU, RSTGI, rows0, rows1, ebuf0, ebuf1, sbuf0, sbuf1,
             bidx, posb, semA0, semA1, semB0, semB1):
    c = lax.axis_index("c")
    s = lax.axis_index("s")
    rows = (rows0, rows1)
    ebuf = (ebuf0, ebuf1)
    sbuf = (sbuf0, sbuf1)
    semA = (semA0, semA1)
    semB = (semB0, semB1)

    def add_off(buf, off, n):
        for j in range(n // L):
            buf[pl.ds(j * L, L)] = buf[pl.ds(j * L, L)] + off

    pltpu.sync_copy(zrows, ACC.at[pl.ds(s * 776, 776)])

    @pl.when(s == 0)
    def _init_user_remap():
        pltpu.sync_copy(initu, RSTGU)

    @pl.when(jnp.logical_and(c == 1, s == 1))
    def _init_item_remap():
        pltpu.sync_copy(initi, RSTGI)

    plsc.subcore_barrier()

    def build(src_hbm, rstg, pos_base):
        def body(g, _):
            pltpu.sync_copy(src_hbm.at[pl.ds(g * 64, 64)], bidx)
            pltpu.sync_copy(pos.at[pl.ds(pos_base + g * 64, 64)], posb)
            pltpu.sync_copy(posb, rstg.at[bidx])
            return 0
        lax.fori_loop(0, B // 64, body, 0)

    @pl.when(s == 0)
    def _build_user_remap():
        build(bu, RSTGU, 0)

    @pl.when(jnp.logical_and(c == 1, s == 1))
    def _build_item_remap():
        build(bp, RSTGI, 0)
        build(bn, RSTGI, B)

    plsc.subcore_barrier()

    def edge_pass(edges, table, rstg, base, nch, slot_off):
        def T(b, off):
            pltpu.sync_copy(edges.at[:, pl.ds(off, 80)], ebuf[b])
            pltpu.sync_copy(rstg.at[ebuf[b].at[0]], sbuf[b])
            if slot_off is not None:
                add_off(sbuf[b], slot_off, 80)

        def Gs(b):
            pltpu.async_copy(table.at[ebuf[b].at[1]], rows[b], semA[b])

        def Gw(b):
            pltpu.make_async_copy(table.at[ebuf[b].at[1]], rows[b],
                                  semA[b]).wait()

        def Ss(b):
            pltpu.async_copy(rows[b], ACC.at[sbuf[b]], semB[b], add=True)

        def Sw(b):
            pltpu.make_async_copy(rows[b], ACC.at[sbuf[b]], semB[b]).wait()

        npairs = nch // 2
        T(0, base)
        Gs(0)

        def body(h, _):
            T(1, base + (2 * h + 1) * 80)
            Gs(1)
            Gw(0)
            Ss(0)
            Sw(0)

            @pl.when(h < npairs - 1)
            def _prime():
                T(0, base + (2 * h + 2) * 80)
                Gs(0)
            Gw(1)
            Ss(1)
            Sw(1)
            return 0
        lax.fori_loop(0, npairs, body, 0)

        if nch % 2 == 1:
            T(0, base + (nch - 1) * 80)
            Gs(0)
            Gw(0)
            Ss(0)
            Sw(0)

    @pl.when(c == 0)
    def _ui_user_pass():
        edge_pass(ui_ei, ia, RSTGU, s * (NCH_UI * 80), NCH_UI, None)

    @pl.when(c == 1)
    def _ui_item_pass():
        edge_pass(ui_ei, ua, RSTGI, s * (NCH_UI * 80), NCH_UI, None)

    uu_off = jnp.where(c == 0, OFF_UU, OFF_UU2).astype(jnp.int32)
    edge_pass(uu_ei, ua, RSTGU, c * (E // NC) + s * (NCH_UU * 80), NCH_UU,
              uu_off)

    plsc.subcore_barrier()

    r64 = rows0.at[pl.ds(0, 64)]
    s64 = sbuf0.at[pl.ds(0, 64)]

    def hbm_gather(src_hbm, o, table, out):
        pltpu.sync_copy(src_hbm.at[pl.ds(o, 64)], bidx)
        pltpu.async_copy(table.at[bidx], r64, semA0).wait()
        pltpu.sync_copy(r64, out.at[pl.ds(o, 64)])

    def acc_gather(rstg, o, off, out):
        pltpu.sync_copy(rstg.at[bidx], s64)
        if off is not None:
            add_off(sbuf0, off, 64)
        pltpu.sync_copy(ACC.at[s64], r64)
        pltpu.sync_copy(r64, out.at[pl.ds(o, 64)])

    @pl.when(c == 0)
    def _final_user_side():
        def fin(k, _):
            o = s * 256 + k * 64
            hbm_gather(bu, o, ua, o_uego)
            acc_gather(RSTGU, o, None, o_uui)
            add_off(sbuf0, OFF_UU, 64)
            pltpu.sync_copy(ACC.at[s64], r64)
            pltpu.sync_copy(r64, o_uuu1.at[pl.ds(o, 64)])
            return 0
        lax.fori_loop(0, 4, fin, 0)

    @pl.when(c == 1)
    def _final_item_side():
        def fin(k, _):
            o = s * 256 + k * 64
            pltpu.sync_copy(bu.at[pl.ds(o, 64)], bidx)
            acc_gather(RSTGU, o, OFF_UU2, o_uuu2)
            hbm_gather(bp, o, ia, o_ip)
            acc_gather(RSTGI, o, None, o_isp)
            hbm_gather(bn, o, ia, o_in)
            acc_gather(RSTGI, o, None, o_isn)
            return 0
        lax.fori_loop(0, 4, fin, 0)


_sc_call = functools.partial(
    pl.kernel,
    out_type=[jax.ShapeDtypeStruct((B, DW), jnp.float32)] * 8,
    mesh=_sc_mesh(),
    compiler_params=pltpu.CompilerParams(
        needs_layout_passes=False, use_tc_tiling_on_sc=False),
    scratch_types=[
        pltpu.VMEM_SHARED((ACC_ROWS, DW), jnp.float32),
        pltpu.VMEM_SHARED((RSZ,), jnp.int32),
        pltpu.VMEM_SHARED((RSZ,), jnp.int32),
        pltpu.VMEM((80, DW), jnp.float32),
        pltpu.VMEM((80, DW), jnp.float32),
        pltpu.VMEM((2, 80), jnp.int32),
        pltpu.VMEM((2, 80), jnp.int32),
        pltpu.VMEM((80,), jnp.int32),
        pltpu.VMEM((80,), jnp.int32),
        pltpu.VMEM((64,), jnp.int32),
        pltpu.VMEM((64,), jnp.int32),
        pltpu.SemaphoreType.DMA,
        pltpu.SemaphoreType.DMA,
        pltpu.SemaphoreType.DMA,
        pltpu.SemaphoreType.DMA,
    ],
)(_sc_body)


def _tc_body(ue, uui, du, uu1, uu2, duu1, duu2, ipr, inr, isp, dp, isn, dn,
             W0, W1, W2, bm, out):
    f32 = jnp.float32
    ue_ = ue[...]
    un = uui[...] / jnp.maximum(du[...], 1.0)
    uu = (uu1[...] + uu2[...]) / jnp.maximum(duu1[...] + duu2[...], 1.0)
    h = (
        jnp.dot(ue_, W0[...], preferred_element_type=f32)
        + jnp.dot(un, W1[...], preferred_element_type=f32)
        + jnp.dot(uu, W2[...], preferred_element_type=f32)
        + bm[...]
    )
    u = jnp.tanh(h)
    ipr_ = ipr[...]
    inr_ = inr[...]
    p = 0.5 * (isp[...] / jnp.maximum(dp[...], 1.0)) + 0.5 * ipr_
    n = 0.5 * (isn[...] / jnp.maximum(dn[...], 1.0)) + 0.5 * inr_
    diff = jnp.sum(u * n, axis=-1) - jnp.sum(u * p, axis=-1)
    cf = jnp.mean(jnp.maximum(diff, 0.0) + jnp.log1p(jnp.exp(-jnp.abs(diff))))
    reg = 0.5 * jnp.mean(
        jnp.sum(ue_ * ue_, axis=-1)
        + jnp.sum(ipr_ * ipr_, axis=-1)
        + jnp.sum(inr_ * inr_, axis=-1)
    )
    out[...] = jnp.reshape(1.0 * cf + 1e-4 * reg, (1, 1))


def kernel(user_emb, item_emb, W_map, b_map, ui_edge_index, uu_edge_index,
           batch_user_pos_neg):
    f32 = jnp.float32
    i32 = jnp.int32

    ones_col = jnp.ones((NU, 1), f32)
    pad_cols = jnp.zeros((NU, DW - D - 1), f32)
    ua = jnp.concatenate([user_emb, ones_col, pad_cols], axis=1)
    ia = jnp.concatenate([item_emb, ones_col, pad_cols], axis=1)

    ui_ei = ui_edge_index.astype(i32)
    uu_ei = uu_edge_index.astype(i32)

    bu = batch_user_pos_neg[:, 0].astype(i32)
    bp = batch_user_pos_neg[:, 1].astype(i32)
    bn = batch_user_pos_neg[:, 2].astype(i32)

    pos = jnp.arange(2 * B, dtype=i32)
    ar = jnp.arange(RSZ, dtype=i32)
    initu = TRASH_U + (ar & 63)
    initi = TRASH_I + (ar & 63)
    zrows = jnp.zeros((776, DW), f32)

    (uego_a, uui_a, uuu1_a, uuu2_a, ip_a, in_a, isp_a, isn_a) = _sc_call(
        ua, ia, ui_ei, uu_ei, bu, bp, bn, pos, initu, initi, zrows)

    def split(a):
        return a[:, :D], a[:, D:D + 1]

    ue_, _ = split(uego_a)
    uui_, du = split(uui_a)
    uu1_, duu1 = split(uuu1_a)
    uu2_, duu2 = split(uuu2_a)
    ipr_, _ = split(ip_a)
    inr_, _ = split(in_a)
    isp_, dp = split(isp_a)
    isn_, dn = split(isn_a)

    W0 = W_map[:D]
    W1 = W_map[D:2 * D]
    W2 = W_map[2 * D:]
    bm = b_map.reshape(1, D)

    out = pl.pallas_call(
        _tc_body,
        out_shape=jax.ShapeDtypeStruct((1, 1), f32),
    )(ue_, uui_, du, uu1_, uu2_, duu1, duu2, ipr_, inr_, isp_, dp, isn_, dn,
      W0, W1, W2, bm)
    return out[0, 0]
